# baseline (speedup 1.0000x reference)
import functools

import jax
import jax.numpy as jnp
from jax import lax
from jax.experimental import pallas as pl
from jax.experimental.pallas import tpu as pltpu
from jax.experimental.pallas import tpu_sc as plsc

N_NODES = 100000
N_SPECIES = 119
TAB_PAD = 128

NUM_CORES = 2
NUM_SUBCORES = 16
NW = NUM_CORES * NUM_SUBCORES

CHUNK = 3136
LAST = N_NODES - (NW - 1) * CHUNK


@functools.partial(
    pl.kernel,
    out_type=jax.ShapeDtypeStruct((N_NODES,), jnp.int32),
    mesh=plsc.VectorSubcoreMesh(core_axis_name="c", subcore_axis_name="s"),
    scratch_types=[
        pltpu.VMEM((TAB_PAD,), jnp.int32),
        pltpu.VMEM((TAB_PAD,), jnp.int32),
        pltpu.VMEM_SHARED((NUM_SUBCORES, TAB_PAD), jnp.int32),
        pltpu.VMEM((CHUNK,), jnp.int32),
        pltpu.VMEM((CHUNK,), jnp.int32),
        pltpu.SemaphoreType.DMA,
    ],
)
def _species_lookup(
    a_hbm, tab_hbm, out_hbm, tab_v, jidx_v, inv_sh, in_v, res_v, sem
):
    c = lax.axis_index("c")
    s = lax.axis_index("s")
    wid = s * NUM_CORES + c

    def run(base, nelem):
        cp = pltpu.async_copy(
            a_hbm.at[pl.ds(base, nelem)], in_v.at[pl.ds(0, nelem)], sem
        )
        pltpu.sync_copy(tab_hbm, tab_v.at[pl.ds(0, N_SPECIES)])
        lane = lax.iota(jnp.int32, 16)
        tail_base = (TAB_PAD // 16 - 1) * 16
        tail = tab_v[pl.ds(tail_base, 16)]
        tail_lane = lane + tail_base
        tab_v[pl.ds(tail_base, 16)] = jnp.where(
            tail_lane < N_SPECIES, tail, tail_lane
        )
        for j in range(TAB_PAD // 16):
            jidx_v[pl.ds(j * 16, 16)] = lane + (j * 16)
        pltpu.sync_copy(jidx_v, inv_sh.at[s].at[tab_v])
        cp.wait()
        pltpu.sync_copy(
            inv_sh.at[s].at[in_v.at[pl.ds(0, nelem)]], res_v.at[pl.ds(0, nelem)]
        )
        pltpu.sync_copy(res_v.at[pl.ds(0, nelem)], out_hbm.at[pl.ds(base, nelem)])

    @pl.when(wid < NW - 1)
    def _():
        run(wid * CHUNK, CHUNK)

    @pl.when(wid == NW - 1)
    def _():
        run((NW - 1) * CHUNK, LAST)


def kernel(atomic_numbers_in, species_table):
    a = atomic_numbers_in.astype(jnp.int32)
    tab = species_table.astype(jnp.int32)
    return _species_lookup(a, tab)


# speedup vs baseline: 1.0175x; 1.0175x over previous
import functools

import jax
import jax.numpy as jnp
from jax import lax
from jax.experimental import pallas as pl
from jax.experimental.pallas import tpu as pltpu
from jax.experimental.pallas import tpu_sc as plsc

N_NODES = 100000
N_SPECIES = 119
TAB_PAD = 128

NUM_CORES = 2
NUM_SUBCORES = 16
NW = NUM_CORES * NUM_SUBCORES

CHUNK = 3136
LAST = N_NODES - (NW - 1) * CHUNK


@functools.partial(
    pl.kernel,
    out_type=jax.ShapeDtypeStruct((N_NODES,), jnp.int32),
    mesh=plsc.VectorSubcoreMesh(core_axis_name="c", subcore_axis_name="s"),
    scratch_types=[
        pltpu.VMEM((TAB_PAD,), jnp.int32),
        pltpu.VMEM((TAB_PAD,), jnp.int32),
        pltpu.VMEM_SHARED((TAB_PAD,), jnp.int32),
        pltpu.VMEM((CHUNK,), jnp.int32),
        pltpu.VMEM((CHUNK,), jnp.int32),
        pltpu.SemaphoreType.DMA,
    ],
)
def _species_lookup(
    a_hbm, tab_hbm, out_hbm, tab_v, jidx_v, inv_sh, in_v, res_v, sem
):
    c = lax.axis_index("c")
    s = lax.axis_index("s")
    wid = s * NUM_CORES + c

    @pl.when(s == 0)
    def _():
        cp = pltpu.async_copy(
            a_hbm.at[pl.ds(wid * CHUNK, CHUNK)], in_v.at[pl.ds(0, CHUNK)], sem
        )
        pltpu.sync_copy(tab_hbm, tab_v.at[pl.ds(0, N_SPECIES)])
        lane = lax.iota(jnp.int32, 16)
        tail_base = (TAB_PAD // 16 - 1) * 16
        tail = tab_v[pl.ds(tail_base, 16)]
        tail_lane = lane + tail_base
        tab_v[pl.ds(tail_base, 16)] = jnp.where(
            tail_lane < N_SPECIES, tail, tail_lane
        )
        for j in range(TAB_PAD // 16):
            jidx_v[pl.ds(j * 16, 16)] = lane + (j * 16)
        pltpu.sync_copy(jidx_v, inv_sh.at[tab_v])
        cp.wait()

    @pl.when((s != 0) & (wid < NW - 1))
    def _():
        pltpu.sync_copy(
            a_hbm.at[pl.ds(wid * CHUNK, CHUNK)], in_v.at[pl.ds(0, CHUNK)]
        )

    @pl.when(wid == NW - 1)
    def _():
        pltpu.sync_copy(
            a_hbm.at[pl.ds((NW - 1) * CHUNK, LAST)], in_v.at[pl.ds(0, LAST)]
        )

    plsc.subcore_barrier()

    def translate(base, nelem):
        pltpu.sync_copy(
            inv_sh.at[in_v.at[pl.ds(0, nelem)]], res_v.at[pl.ds(0, nelem)]
        )
        pltpu.sync_copy(res_v.at[pl.ds(0, nelem)], out_hbm.at[pl.ds(base, nelem)])

    @pl.when(wid < NW - 1)
    def _():
        translate(wid * CHUNK, CHUNK)

    @pl.when(wid == NW - 1)
    def _():
        translate((NW - 1) * CHUNK, LAST)


def kernel(atomic_numbers_in, species_table):
    a = atomic_numbers_in.astype(jnp.int32)
    tab = species_table.astype(jnp.int32)
    return _species_lookup(a, tab)


# speedup vs baseline: 1.0179x; 1.0005x over previous
import functools

import jax
import jax.numpy as jnp
from jax import lax
from jax.experimental import pallas as pl
from jax.experimental.pallas import tpu as pltpu
from jax.experimental.pallas import tpu_sc as plsc

N_NODES = 100000
N_SPECIES = 119
TAB_PAD = 128

NUM_CORES = 2
NUM_SUBCORES = 16
NW = NUM_CORES * NUM_SUBCORES

CHUNK = 3136
LAST = N_NODES - (NW - 1) * CHUNK


@functools.partial(
    pl.kernel,
    out_type=jax.ShapeDtypeStruct((N_NODES,), jnp.int32),
    mesh=plsc.VectorSubcoreMesh(core_axis_name="c", subcore_axis_name="s"),
    scratch_types=[
        pltpu.VMEM((TAB_PAD,), jnp.int32),
        pltpu.VMEM((TAB_PAD,), jnp.int32),
        pltpu.VMEM_SHARED((TAB_PAD,), jnp.int32),
        pltpu.VMEM((CHUNK,), jnp.int32),
        pltpu.VMEM((CHUNK,), jnp.int32),
        pltpu.SemaphoreType.DMA,
        pltpu.SemaphoreType.DMA,
        pltpu.SemaphoreType.DMA,
    ],
)
def _species_lookup(
    a_hbm, tab_hbm, out_hbm, tab_v, jidx_v, inv_sh, in_v, res_v, sem, sem2, sem3
):
    c = lax.axis_index("c")
    s = lax.axis_index("s")
    wid = s * NUM_CORES + c

    @pl.when(s == 0)
    def _():
        cp = pltpu.async_copy(
            a_hbm.at[pl.ds(wid * CHUNK, CHUNK)], in_v.at[pl.ds(0, CHUNK)], sem
        )
        pltpu.sync_copy(tab_hbm, tab_v.at[pl.ds(0, N_SPECIES)])
        lane = lax.iota(jnp.int32, 16)
        tail_base = (TAB_PAD // 16 - 1) * 16
        tail = tab_v[pl.ds(tail_base, 16)]
        tail_lane = lane + tail_base
        tab_v[pl.ds(tail_base, 16)] = jnp.where(
            tail_lane < N_SPECIES, tail, tail_lane
        )
        for j in range(TAB_PAD // 16):
            jidx_v[pl.ds(j * 16, 16)] = lane + (j * 16)
        pltpu.sync_copy(jidx_v, inv_sh.at[tab_v])
        cp.wait()

    @pl.when((s != 0) & (wid < NW - 1))
    def _():
        pltpu.sync_copy(
            a_hbm.at[pl.ds(wid * CHUNK, CHUNK)], in_v.at[pl.ds(0, CHUNK)]
        )

    @pl.when(wid == NW - 1)
    def _():
        pltpu.sync_copy(
            a_hbm.at[pl.ds((NW - 1) * CHUNK, LAST)], in_v.at[pl.ds(0, LAST)]
        )

    plsc.subcore_barrier()

    def translate(base, nelem):
        h = nelem // 2
        g0 = pltpu.async_copy(
            inv_sh.at[in_v.at[pl.ds(0, h)]], res_v.at[pl.ds(0, h)], sem
        )
        g1 = pltpu.async_copy(
            inv_sh.at[in_v.at[pl.ds(h, nelem - h)]],
            res_v.at[pl.ds(h, nelem - h)],
            sem2,
        )
        g0.wait()
        o0 = pltpu.async_copy(
            res_v.at[pl.ds(0, h)], out_hbm.at[pl.ds(base, h)], sem3
        )
        g1.wait()
        pltpu.sync_copy(
            res_v.at[pl.ds(h, nelem - h)], out_hbm.at[pl.ds(base + h, nelem - h)]
        )
        o0.wait()

    @pl.when(wid < NW - 1)
    def _():
        translate(wid * CHUNK, CHUNK)

    @pl.when(wid == NW - 1)
    def _():
        translate((NW - 1) * CHUNK, LAST)


def kernel(atomic_numbers_in, species_table):
    a = atomic_numbers_in.astype(jnp.int32)
    tab = species_table.astype(jnp.int32)
    return _species_lookup(a, tab)


# speedup vs baseline: 1.0248x; 1.0068x over previous
import functools

import jax
import jax.numpy as jnp
from jax import lax
from jax.experimental import pallas as pl
from jax.experimental.pallas import tpu as pltpu
from jax.experimental.pallas import tpu_sc as plsc

N_NODES = 100000
N_SPECIES = 119
TAB_PAD = 128

NUM_CORES = 1
NUM_SUBCORES = 16
NW = NUM_CORES * NUM_SUBCORES

CHUNK = 6272
LAST = N_NODES - (NW - 1) * CHUNK


@functools.partial(
    pl.kernel,
    out_type=jax.ShapeDtypeStruct((N_NODES,), jnp.int32),
    mesh=plsc.VectorSubcoreMesh(core_axis_name="c", subcore_axis_name="s", num_cores=1),
    scratch_types=[
        pltpu.VMEM((TAB_PAD,), jnp.int32),
        pltpu.VMEM((TAB_PAD,), jnp.int32),
        pltpu.VMEM_SHARED((TAB_PAD,), jnp.int32),
        pltpu.VMEM((CHUNK,), jnp.int32),
        pltpu.VMEM((CHUNK,), jnp.int32),
        pltpu.SemaphoreType.DMA,
        pltpu.SemaphoreType.DMA,
        pltpu.SemaphoreType.DMA,
    ],
)
def _species_lookup(
    a_hbm, tab_hbm, out_hbm, tab_v, jidx_v, inv_sh, in_v, res_v, sem, sem2, sem3
):
    c = lax.axis_index("c")
    s = lax.axis_index("s")
    wid = s * NUM_CORES + c

    @pl.when(s == 0)
    def _():
        cp = pltpu.async_copy(
            a_hbm.at[pl.ds(wid * CHUNK, CHUNK)], in_v.at[pl.ds(0, CHUNK)], sem
        )
        pltpu.sync_copy(tab_hbm, tab_v.at[pl.ds(0, N_SPECIES)])
        lane = lax.iota(jnp.int32, 16)
        tail_base = (TAB_PAD // 16 - 1) * 16
        tail = tab_v[pl.ds(tail_base, 16)]
        tail_lane = lane + tail_base
        tab_v[pl.ds(tail_base, 16)] = jnp.where(
            tail_lane < N_SPECIES, tail, tail_lane
        )
        for j in range(TAB_PAD // 16):
            jidx_v[pl.ds(j * 16, 16)] = lane + (j * 16)
        pltpu.sync_copy(jidx_v, inv_sh.at[tab_v])
        cp.wait()

    @pl.when((s != 0) & (wid < NW - 1))
    def _():
        pltpu.sync_copy(
            a_hbm.at[pl.ds(wid * CHUNK, CHUNK)], in_v.at[pl.ds(0, CHUNK)]
        )

    @pl.when(wid == NW - 1)
    def _():
        pltpu.sync_copy(
            a_hbm.at[pl.ds((NW - 1) * CHUNK, LAST)], in_v.at[pl.ds(0, LAST)]
        )

    plsc.subcore_barrier()

    def translate(base, nelem):
        h = nelem // 2
        g0 = pltpu.async_copy(
            inv_sh.at[in_v.at[pl.ds(0, h)]], res_v.at[pl.ds(0, h)], sem
        )
        g1 = pltpu.async_copy(
            inv_sh.at[in_v.at[pl.ds(h, nelem - h)]],
            res_v.at[pl.ds(h, nelem - h)],
            sem2,
        )
        g0.wait()
        o0 = pltpu.async_copy(
            res_v.at[pl.ds(0, h)], out_hbm.at[pl.ds(base, h)], sem3
        )
        g1.wait()
        pltpu.sync_copy(
            res_v.at[pl.ds(h, nelem - h)], out_hbm.at[pl.ds(base + h, nelem - h)]
        )
        o0.wait()

    @pl.when(wid < NW - 1)
    def _():
        translate(wid * CHUNK, CHUNK)

    @pl.when(wid == NW - 1)
    def _():
        translate((NW - 1) * CHUNK, LAST)


def kernel(atomic_numbers_in, species_table):
    a = atomic_numbers_in.astype(jnp.int32)
    tab = species_table.astype(jnp.int32)
    return _species_lookup(a, tab)
